# baseline (speedup 1.0000x reference)
import math

import jax
import jax.numpy as jnp
from jax import lax
from jax.experimental import pallas as pl
from jax.experimental.pallas import tpu as pltpu
from jax.experimental.pallas import tpu_sc as plsc

EMB_D = 32
SCALE = math.sqrt(float(EMB_D))

NC, NS, LANES = 2, 16, 16
NW = NC * NS

N_SEQ, N_TOK = 16384, 50
B_TOTAL = N_SEQ * N_TOK
GIDX = 128
NBLK = B_TOTAL // GIDX
BPW = NBLK // NW
SB = N_SEQ // GIDX


def _emb_body(idx_hbm, table_hbm, out_hbm,
              idx_v, rows0, rows1, t40, t41, gsem0, gsem1, ssem0, ssem1):
    wid = lax.axis_index("s") * NC + lax.axis_index("c")
    g0 = wid * BPW

    pltpu.sync_copy(idx_hbm.at[pl.ds(g0, BPW)], idx_v)

    lane = lax.iota(jnp.int32, LANES)

    def issue_gather(i, rows_b, gsem_b):
        pltpu.async_copy(table_hbm.at[idx_v.at[i]], rows_b, gsem_b)

    def wait_gather(i, rows_b, gsem_b):
        pltpu.make_async_copy(table_hbm.at[idx_v.at[i]], rows_b, gsem_b).wait()

    def transpose_scale(rows_b, t4_b):
        for kk in range(GIDX // LANES):
            rvec = lane + (kk * LANES)
            for dt in range(4):
                for dp in range(8):
                    d = dt * 8 + dp
                    cvec = jnp.full((LANES,), d, dtype=jnp.int32)
                    v = plsc.load_gather(rows_b, [rvec, cvec])
                    t4_b[dt, dp, pl.ds(kk * LANES, LANES)] = v * SCALE

    def issue_stores(j, st, t4_b, ssem_b):
        for dt in range(4):
            pltpu.async_copy(t4_b.at[dt], out_hbm.at[j, dt, st], ssem_b)

    def wait_stores(j, st, t4_b, ssem_b):
        for dt in range(4):
            pltpu.make_async_copy(t4_b.at[dt], out_hbm.at[j, dt, st], ssem_b
                                  ).wait()

    issue_gather(0, rows0, gsem0)
    issue_gather(1, rows1, gsem1)

    bufs = ((rows0, t40, gsem0, ssem0), (rows1, t41, gsem1, ssem1))

    def pair(t, _):
        for b in range(2):
            rows_b, t4_b, gsem_b, ssem_b = bufs[b]
            i = 2 * t + b
            g = g0 + i
            j = g >> 7
            st = g & (SB - 1)
            wait_gather(i, rows_b, gsem_b)

            @pl.when(t > 0)
            def _():
                g_prev = g - 2
                wait_stores(g_prev >> 7, g_prev & (SB - 1), t4_b, ssem_b)

            transpose_scale(rows_b, t4_b)

            @pl.when(i + 2 < BPW)
            def _():
                issue_gather(i + 2, rows_b, gsem_b)

            issue_stores(j, st, t4_b, ssem_b)
        return 0

    lax.fori_loop(0, BPW // 2, pair, 0)

    for b in range(2):
        rows_b, t4_b, gsem_b, ssem_b = bufs[b]
        g = g0 + BPW - 2 + b
        wait_stores(g >> 7, g & (SB - 1), t4_b, ssem_b)


@jax.jit
def _emb(idx2, table):
    mesh = plsc.VectorSubcoreMesh(core_axis_name="c", subcore_axis_name="s")
    f = pl.kernel(
        _emb_body,
        out_type=jax.ShapeDtypeStruct((N_TOK, 4, SB, 8, GIDX), jnp.float32),
        mesh=mesh,
        scratch_types=[
            pltpu.VMEM((BPW, GIDX), jnp.int32),
            pltpu.VMEM((GIDX, EMB_D), jnp.float32),
            pltpu.VMEM((GIDX, EMB_D), jnp.float32),
            pltpu.VMEM((4, 8, GIDX), jnp.float32),
            pltpu.VMEM((4, 8, GIDX), jnp.float32),
            pltpu.SemaphoreType.DMA,
            pltpu.SemaphoreType.DMA,
            pltpu.SemaphoreType.DMA,
            pltpu.SemaphoreType.DMA,
        ],
        compiler_params=pltpu.CompilerParams(
            use_tc_tiling_on_sc=False, needs_layout_passes=False),
    )
    return f(idx2, table)


def kernel(x, embed_table):
    idx2 = x.T.reshape(NBLK, GIDX).astype(jnp.int32)
    out5 = _emb(idx2, embed_table)
    return out5.transpose(2, 4, 0, 1, 3).reshape(N_SEQ, N_TOK, EMB_D)


# speedup vs baseline: 1.0724x; 1.0724x over previous
import math

import jax
import jax.numpy as jnp
from jax import lax
from jax.experimental import pallas as pl
from jax.experimental.pallas import tpu as pltpu
from jax.experimental.pallas import tpu_sc as plsc

EMB_D = 32
SCALE = math.sqrt(float(EMB_D))

NC, NS, LANES = 2, 16, 16
NW = NC * NS

N_SEQ, N_TOK = 16384, 50
B_TOTAL = N_SEQ * N_TOK
GIDX = 128
NBLK = B_TOTAL // GIDX
BPW = NBLK // NW
SB = N_SEQ // GIDX


def _emb_body(idx_hbm, table_hbm, out_hbm,
              idx_v, rows0, rows1, t40, t41, gsem0, gsem1, ssem0, ssem1):
    wid = lax.axis_index("s") * NC + lax.axis_index("c")
    g0 = wid * BPW

    pltpu.sync_copy(idx_hbm.at[pl.ds(g0, BPW)], idx_v)

    lane = lax.iota(jnp.int32, LANES)

    def issue_gather(i, rows_b, gsem_b):
        pltpu.async_copy(table_hbm.at[idx_v.at[i]], rows_b, gsem_b)

    def wait_gather(i, rows_b, gsem_b):
        pltpu.make_async_copy(table_hbm.at[idx_v.at[i]], rows_b, gsem_b).wait()

    def transpose_scale(rows_b, t4_b):
        def body(kk, _):
            rvec = lane + kk * LANES
            base = kk * LANES
            for dt in range(4):
                for dp in range(8):
                    d = dt * 8 + dp
                    cvec = jnp.full((LANES,), d, dtype=jnp.int32)
                    v = plsc.load_gather(rows_b, [rvec, cvec])
                    t4_b[dt, dp, pl.ds(base, LANES)] = v * SCALE
            return 0

        lax.fori_loop(0, GIDX // LANES, body, 0)

    def issue_stores(j, st, t4_b, ssem_b):
        for dt in range(4):
            pltpu.async_copy(t4_b.at[dt], out_hbm.at[j, dt, st], ssem_b)

    def wait_stores(j, st, t4_b, ssem_b):
        for dt in range(4):
            pltpu.make_async_copy(t4_b.at[dt], out_hbm.at[j, dt, st], ssem_b
                                  ).wait()

    issue_gather(0, rows0, gsem0)
    issue_gather(1, rows1, gsem1)

    bufs = ((rows0, t40, gsem0, ssem0), (rows1, t41, gsem1, ssem1))

    def pair(t, _):
        for b in range(2):
            rows_b, t4_b, gsem_b, ssem_b = bufs[b]
            i = 2 * t + b
            g = g0 + i
            j = g >> 7
            st = g & (SB - 1)
            wait_gather(i, rows_b, gsem_b)

            @pl.when(t > 0)
            def _():
                g_prev = g - 2
                wait_stores(g_prev >> 7, g_prev & (SB - 1), t4_b, ssem_b)

            transpose_scale(rows_b, t4_b)

            @pl.when(i + 2 < BPW)
            def _():
                issue_gather(i + 2, rows_b, gsem_b)

            issue_stores(j, st, t4_b, ssem_b)
        return 0

    lax.fori_loop(0, BPW // 2, pair, 0)

    for b in range(2):
        rows_b, t4_b, gsem_b, ssem_b = bufs[b]
        g = g0 + BPW - 2 + b
        wait_stores(g >> 7, g & (SB - 1), t4_b, ssem_b)


@jax.jit
def _emb(idx2, table):
    mesh = plsc.VectorSubcoreMesh(core_axis_name="c", subcore_axis_name="s")
    f = pl.kernel(
        _emb_body,
        out_type=jax.ShapeDtypeStruct((N_TOK, 4, SB, 8, GIDX), jnp.float32),
        mesh=mesh,
        scratch_types=[
            pltpu.VMEM((BPW, GIDX), jnp.int32),
            pltpu.VMEM((GIDX, EMB_D), jnp.float32),
            pltpu.VMEM((GIDX, EMB_D), jnp.float32),
            pltpu.VMEM((4, 8, GIDX), jnp.float32),
            pltpu.VMEM((4, 8, GIDX), jnp.float32),
            pltpu.SemaphoreType.DMA,
            pltpu.SemaphoreType.DMA,
            pltpu.SemaphoreType.DMA,
            pltpu.SemaphoreType.DMA,
        ],
        compiler_params=pltpu.CompilerParams(
            use_tc_tiling_on_sc=False, needs_layout_passes=False),
    )
    return f(idx2, table)


def kernel(x, embed_table):
    idx2 = x.T.reshape(NBLK, GIDX).astype(jnp.int32)
    out5 = _emb(idx2, embed_table)
    return out5.transpose(2, 4, 0, 1, 3).reshape(N_SEQ, N_TOK, EMB_D)


# speedup vs baseline: 1.2414x; 1.1575x over previous
import math

import jax
import jax.numpy as jnp
from jax import lax
from jax.experimental import pallas as pl
from jax.experimental.pallas import tpu as pltpu
from jax.experimental.pallas import tpu_sc as plsc

EMB_D = 32
SCALE = math.sqrt(float(EMB_D))

NC, NS, LANES = 2, 16, 16
NW = NC * NS

N_SEQ, N_TOK = 16384, 50
B_TOTAL = N_SEQ * N_TOK
GIDX = 128
NBLK = B_TOTAL // GIDX
BPW = NBLK // NW
SB = N_SEQ // GIDX
TILE_W = 8 * GIDX
UNROLL = 8


def _emb_body(idx_hbm, table_hbm, out_hbm,
              idx_v, rows0, rows1, t40, t41, gsem0, gsem1, ssem0, ssem1):
    wid = lax.axis_index("s") * NC + lax.axis_index("c")
    g0 = wid * BPW

    pltpu.sync_copy(idx_hbm.at[pl.ds(g0, BPW)], idx_v)

    lane = lax.iota(jnp.int32, LANES)
    pat_lo = [(lane >> 3) * TILE_W + (lane & 7) * GIDX + r
              for r in range(UNROLL)]
    pat_hi = [p + 2 * TILE_W for p in pat_lo]

    def issue_gather(i, rows_b, gsem_b):
        pltpu.async_copy(table_hbm.at[idx_v.at[i]], rows_b, gsem_b)

    def wait_gather(i, rows_b, gsem_b):
        pltpu.make_async_copy(table_hbm.at[idx_v.at[i]], rows_b, gsem_b).wait()

    def transpose_scale(rows_b, t4_b):
        def body(it, _):
            sp0 = it * UNROLL
            spv = jnp.full((LANES,), sp0, dtype=jnp.int32)
            for r in range(UNROLL):
                sp = sp0 + r
                lo = rows_b[sp, pl.ds(0, LANES)] * SCALE
                hi = rows_b[sp, pl.ds(LANES, LANES)] * SCALE
                plsc.store_scatter(t4_b, [pat_lo[r] + spv], lo)
                plsc.store_scatter(t4_b, [pat_hi[r] + spv], hi)
            return 0

        lax.fori_loop(0, GIDX // UNROLL, body, 0)

    def issue_stores(j, st, t4_b, ssem_b):
        for dt in range(4):
            pltpu.async_copy(t4_b.at[pl.ds(dt * TILE_W, TILE_W)],
                             out_hbm.at[j, dt, st], ssem_b)

    def wait_stores(j, st, t4_b, ssem_b):
        for dt in range(4):
            pltpu.make_async_copy(t4_b.at[pl.ds(dt * TILE_W, TILE_W)],
                                  out_hbm.at[j, dt, st], ssem_b).wait()

    issue_gather(0, rows0, gsem0)
    issue_gather(1, rows1, gsem1)

    bufs = ((rows0, t40, gsem0, ssem0), (rows1, t41, gsem1, ssem1))

    def pair(t, _):
        for b in range(2):
            rows_b, t4_b, gsem_b, ssem_b = bufs[b]
            i = 2 * t + b
            g = g0 + i
            j = g >> 7
            st = g & (SB - 1)
            wait_gather(i, rows_b, gsem_b)

            @pl.when(t > 0)
            def _():
                g_prev = g - 2
                wait_stores(g_prev >> 7, g_prev & (SB - 1), t4_b, ssem_b)

            transpose_scale(rows_b, t4_b)

            @pl.when(i + 2 < BPW)
            def _():
                issue_gather(i + 2, rows_b, gsem_b)

            issue_stores(j, st, t4_b, ssem_b)
        return 0

    lax.fori_loop(0, BPW // 2, pair, 0)

    for b in range(2):
        rows_b, t4_b, gsem_b, ssem_b = bufs[b]
        g = g0 + BPW - 2 + b
        wait_stores(g >> 7, g & (SB - 1), t4_b, ssem_b)


@jax.jit
def _emb(idx2, table):
    mesh = plsc.VectorSubcoreMesh(core_axis_name="c", subcore_axis_name="s")
    f = pl.kernel(
        _emb_body,
        out_type=jax.ShapeDtypeStruct((N_TOK, 4, SB, TILE_W), jnp.float32),
        mesh=mesh,
        scratch_types=[
            pltpu.VMEM((BPW, GIDX), jnp.int32),
            pltpu.VMEM((GIDX, EMB_D), jnp.float32),
            pltpu.VMEM((GIDX, EMB_D), jnp.float32),
            pltpu.VMEM((4 * TILE_W,), jnp.float32),
            pltpu.VMEM((4 * TILE_W,), jnp.float32),
            pltpu.SemaphoreType.DMA,
            pltpu.SemaphoreType.DMA,
            pltpu.SemaphoreType.DMA,
            pltpu.SemaphoreType.DMA,
        ],
        compiler_params=pltpu.CompilerParams(
            use_tc_tiling_on_sc=False, needs_layout_passes=False),
    )
    return f(idx2, table)


def kernel(x, embed_table):
    idx2 = x.T.reshape(NBLK, GIDX).astype(jnp.int32)
    out5 = _emb(idx2, embed_table)
    return (out5.reshape(N_TOK, 4, SB, 8, GIDX)
            .transpose(2, 4, 0, 1, 3)
            .reshape(N_SEQ, N_TOK, EMB_D))
